# baseline (speedup 1.0000x reference)
import functools

import jax
import jax.numpy as jnp
import numpy as np
from jax import lax
from jax.experimental import pallas as pl
from jax.experimental.pallas import tpu as pltpu
from jax.experimental.pallas import tpu_sc as plsc

_NX, _NY = 392, 120
_N0 = _NX * _NY
_N1 = (_NX // 2) * (_NY // 2)
_N2 = (_NX // 4) * (_NY // 4)
_NP0, _NP1, _NP2 = 47104, 11776, 2944
_N12 = _NP1 + _NP2
_M0, _M1, _M2 = _NP0 // 4, _NP1 // 4, _NP2 // 4
_CPE = 128
_EP0, _K0 = 188416, 46
_EP1, _K1 = 49152, 12
_EP2, _K2 = 16384, 4
_KD = _K1 + _K2
_R4 = 736


def _mesh():
    return plsc.VectorSubcoreMesh(core_axis_name="c", subcore_axis_name="s")


_SC_PARAMS = pltpu.CompilerParams(use_tc_tiling_on_sc=False)


def _sc_deg(col0, col12):
    rpt0 = _NP0 // 16
    rpt12 = _N12 // 16
    zr = 736
    nz0 = rpt0 // zr

    @functools.partial(
        pl.kernel,
        out_type=[jax.ShapeDtypeStruct((2, _NP0, 16), jnp.float32),
                  jax.ShapeDtypeStruct((2, _N12, 16), jnp.float32)],
        mesh=_mesh(),
        compiler_params=_SC_PARAMS,
        scratch_types=[
            pltpu.VMEM((_K0, _CPE), jnp.int32),
            pltpu.VMEM((_KD, _CPE), jnp.int32),
            pltpu.VMEM((_CPE, 16), jnp.float32),
            pltpu.VMEM((rpt12, 16), jnp.float32),
            pltpu.VMEM_SHARED((_NP0, 16), jnp.float32),
            pltpu.VMEM_SHARED((_N12, 16), jnp.float32),
            pltpu.SemaphoreType.DMA,
        ],
    )
    def kfn(col0_hbm, col12_hbm, out0_hbm, out12_hbm, coli0, coli12,
            ones16, zbuf16, dacc0, dacc12, sem):
        cid = lax.axis_index("c")
        sid = lax.axis_index("s")
        wid = cid * 16 + sid

        def fill16(i, _):
            zbuf16[i, :] = jnp.zeros((16,), jnp.float32)
            return _

        lax.fori_loop(0, rpt12, fill16, None)

        def fillo16(i, _):
            ones16[i, :] = jnp.ones((16,), jnp.float32)
            return _

        lax.fori_loop(0, _CPE, fillo16, None)

        def zc0(i, _):
            pltpu.sync_copy(zbuf16.at[pl.ds(0, zr)],
                            dacc0.at[pl.ds(sid * rpt0 + i * zr, zr)])
            return _

        lax.fori_loop(0, nz0, zc0, None)
        pltpu.sync_copy(zbuf16, dacc12.at[pl.ds(sid * rpt12, rpt12)])
        pltpu.sync_copy(col0_hbm.at[wid], coli0)
        pltpu.sync_copy(col12_hbm.at[wid], coli12)
        plsc.subcore_barrier()

        def edge0(j, _):
            pltpu.sync_copy(ones16, dacc0.at[coli0.at[j]], add=True)
            return _

        lax.fori_loop(0, _K0, edge0, None)

        def edge12(j, _):
            pltpu.sync_copy(ones16, dacc12.at[coli12.at[j]], add=True)
            return _

        lax.fori_loop(0, _KD, edge12, None)
        plsc.subcore_barrier()
        pltpu.sync_copy(dacc0.at[pl.ds(sid * rpt0, rpt0)],
                        out0_hbm.at[cid, pl.ds(sid * rpt0, rpt0)])
        pltpu.sync_copy(dacc12.at[pl.ds(sid * rpt12, rpt12)],
                        out12_hbm.at[cid, pl.ds(sid * rpt12, rpt12)])

    return kfn(col0, col12)


def _sc_conv(s, row3, col3, np_rows):
    k_ops = row3.shape[1]
    cpe = row3.shape[2]
    khalf = k_ops // 2
    rpt = np_rows // 16
    zr = 184
    nzero = rpt // zr

    @functools.partial(
        pl.kernel,
        out_type=jax.ShapeDtypeStruct((2, np_rows, 32), jnp.float32),
        mesh=_mesh(),
        compiler_params=_SC_PARAMS,
        scratch_types=[
            pltpu.VMEM((k_ops, cpe), jnp.int32),
            pltpu.VMEM((k_ops, cpe), jnp.int32),
            pltpu.VMEM((cpe, 32), jnp.float32),
            pltpu.VMEM((cpe, 32), jnp.float32),
            pltpu.VMEM((zr, 32), jnp.float32),
            pltpu.VMEM_SHARED((np_rows, 32), jnp.float32),
            pltpu.SemaphoreType.DMA,
            pltpu.SemaphoreType.DMA,
            pltpu.SemaphoreType.DMA,
            pltpu.SemaphoreType.DMA,
            pltpu.SemaphoreType.DMA,
        ],
    )
    def kfn(s_hbm, row_hbm, col_hbm, out_hbm, rowi, coli, buf0, buf1, zbuf,
            acc, semz, sem0, sem1, sems0, sems1):
        cid = lax.axis_index("c")
        sid = lax.axis_index("s")
        wid = cid * 16 + sid

        def fill(i, _):
            zbuf[i, pl.ds(0, 16)] = jnp.zeros((16,), jnp.float32)
            zbuf[i, pl.ds(16, 16)] = jnp.zeros((16,), jnp.float32)
            return _

        lax.fori_loop(0, zr, fill, None)

        pltpu.async_copy(row_hbm.at[wid], rowi, semz)
        pltpu.async_copy(col_hbm.at[wid], coli, semz)

        def zc(i, _):
            pltpu.sync_copy(zbuf, acc.at[pl.ds(sid * rpt + i * zr, zr)])
            return _

        lax.fori_loop(0, nzero, zc, None)
        pltpu.make_async_copy(row_hbm.at[wid], rowi, semz).wait()
        pltpu.make_async_copy(col_hbm.at[wid], coli, semz).wait()
        plsc.subcore_barrier()

        pltpu.async_copy(s_hbm.at[rowi.at[0]], buf0, sem0)

        def edge(jj, _):
            j = jj * 2

            @pl.when(jj > 0)
            def _wait_s1():
                pltpu.make_async_copy(buf1, acc.at[coli.at[j - 1]],
                                      sems1).wait()

            pltpu.async_copy(s_hbm.at[rowi.at[j + 1]], buf1, sem1)
            pltpu.make_async_copy(s_hbm.at[rowi.at[j]], buf0, sem0).wait()
            pltpu.async_copy(buf0, acc.at[coli.at[j]], sems0, add=True)

            @pl.when(j + 2 < k_ops)
            def _prefetch():
                pltpu.make_async_copy(buf0, acc.at[coli.at[j]], sems0).wait()
                pltpu.async_copy(s_hbm.at[rowi.at[j + 2]], buf0, sem0)

            pltpu.make_async_copy(s_hbm.at[rowi.at[j + 1]], buf1, sem1).wait()
            pltpu.async_copy(buf1, acc.at[coli.at[j + 1]], sems1, add=True)
            return _

        lax.fori_loop(0, khalf, edge, None)
        pltpu.make_async_copy(buf0, acc.at[coli.at[k_ops - 2]], sems0).wait()
        pltpu.make_async_copy(buf1, acc.at[coli.at[k_ops - 1]], sems1).wait()
        plsc.subcore_barrier()
        pltpu.sync_copy(acc.at[pl.ds(sid * rpt, rpt)],
                        out_hbm.at[cid, pl.ds(sid * rpt, rpt)])

    return kfn(s, row3, col3)


def _rows(c):
    return pl.BlockSpec((_R4, c), lambda i: (i, 0))


def _const(shape):
    return pl.BlockSpec(shape, lambda i: tuple(0 for _ in shape))


def _f32(shape):
    return jax.ShapeDtypeStruct(shape, jnp.float32)


_XE = np.zeros((128, 128), np.float32)
_XO = np.zeros((128, 128), np.float32)
for _i in range(4):
    for _c in range(32):
        _XE[16 * _i, 32 * _i + _c] = 1.0
        _XO[64 + 16 * _i, 32 * _i + _c] = 1.0


def _tc_dinv12(dp, xe, xo):
    m_rows = _N12 // 4
    qin = _R4 // 2

    def body(d_ref, xe_ref, xo_ref, o_ref):
        deg = d_ref[0] + d_ref[1]
        t = jnp.broadcast_to(deg[:, None, :], (qin, 2, 128)).reshape(_R4, 128)
        riota = lax.broadcasted_iota(jnp.int32, (_R4, 128), 0)
        val = jnp.where(
            riota % 2 == 0,
            jnp.dot(t, xe_ref[...], preferred_element_type=jnp.float32, precision=lax.Precision.HIGHEST),
            jnp.dot(t, xo_ref[...], preferred_element_type=jnp.float32, precision=lax.Precision.HIGHEST))
        o_ref[...] = lax.rsqrt(1.0 + val)

    return pl.pallas_call(
        body,
        grid=(m_rows // _R4,),
        in_specs=[pl.BlockSpec((2, qin, 128), lambda i: (0, i, 0)),
                  _const((128, 128)), _const((128, 128))],
        out_specs=_rows(128),
        out_shape=_f32((m_rows, 128)),
    )(dp, xe, xo)


def _tc_head(xp, dp0, fc1_Wb, fc1_bt, W1b, xe, xo):
    qin = _R4 // 2

    def body(x_ref, d_ref, fw_ref, fb_ref, w1_ref, xe_ref, xo_ref,
             s_ref, di_ref):
        deg = d_ref[0] + d_ref[1]
        t = jnp.broadcast_to(deg[:, None, :], (qin, 2, 128)).reshape(_R4, 128)
        riota = lax.broadcasted_iota(jnp.int32, (_R4, 128), 0)
        val = jnp.where(
            riota % 2 == 0,
            jnp.dot(t, xe_ref[...], preferred_element_type=jnp.float32, precision=lax.Precision.HIGHEST),
            jnp.dot(t, xo_ref[...], preferred_element_type=jnp.float32, precision=lax.Precision.HIGHEST))
        dinv = lax.rsqrt(1.0 + val)
        h = jnp.dot(x_ref[...], fw_ref[...], preferred_element_type=jnp.float32, precision=lax.Precision.HIGHEST)
        h = jnp.maximum(h + fb_ref[...], 0.0)
        s_ref[...] = jnp.dot(h, w1_ref[...], preferred_element_type=jnp.float32, precision=lax.Precision.HIGHEST) * dinv
        di_ref[...] = dinv

    return pl.pallas_call(
        body,
        grid=(_M0 // _R4,),
        in_specs=[_rows(16), pl.BlockSpec((2, qin, 128), lambda i: (0, i, 0)),
                  _const((16, 128)), _const((1, 128)), _const((128, 128)),
                  _const((128, 128)), _const((128, 128))],
        out_specs=[_rows(128), _rows(128)],
        out_shape=[_f32((_M0, 128)), _f32((_M0, 128))],
    )(xp, dp0, fc1_Wb, fc1_bt, W1b, xe, xo)


def _tc_scale(inp, dinv, Wb, m_rows):
    def body(x_ref, di_ref, w_ref, s_ref):
        s_ref[...] = jnp.dot(x_ref[...], w_ref[...], preferred_element_type=jnp.float32, precision=lax.Precision.HIGHEST) * di_ref[...]

    return pl.pallas_call(
        body,
        grid=(m_rows // _R4,),
        in_specs=[_rows(128), _rows(128), _const((128, 128))],
        out_specs=_rows(128),
        out_shape=_f32((m_rows, 128)),
    )(inp, dinv, Wb)


_EJ = np.zeros((4, 128, 128), np.float32)
_OJ = np.zeros((4, 128, 128), np.float32)
for _j in range(4):
    for _b in (0, 1):
        for _u in (0, 1):
            for _cc in range(16):
                _EJ[_j, 32 * _j + 16 * _b + _cc, 64 * _u + 32 * _b + 2 * _cc] = 1.0
                _OJ[_j, 32 * _j + 16 * _b + _cc, 64 * _u + 32 * _b + 2 * _cc + 1] = 1.0


def _tc_up_res_scale(a, c, dinv, Wb, EW, OW, m_out, aligned, off=0):
    q = m_out // 8

    def body(a_ref, c_ref, di_ref, w_ref, ew_ref, ow_ref, s_ref):
        xa = c_ref[0:q, :]
        if aligned:
            xb = c_ref[off:off + q, :]
        else:
            xb = jnp.concatenate(
                [c_ref[q - 1:2 * q - 1, 64:128], c_ref[q:2 * q, 0:64]],
                axis=1)
        ys = [
            jnp.dot(xa, ew_ref[j], preferred_element_type=jnp.float32, precision=lax.Precision.HIGHEST)
            + jnp.dot(xb, ow_ref[j], preferred_element_type=jnp.float32, precision=lax.Precision.HIGHEST)
            for j in range(4)
        ]
        st = jnp.stack(ys, axis=1)
        st = jnp.broadcast_to(st[:, :, None, :], (q, 4, 2, 128))
        up = st.reshape(m_out, 128)
        base = jnp.dot(a_ref[...], w_ref[...], preferred_element_type=jnp.float32, precision=lax.Precision.HIGHEST)
        s_ref[...] = (base + up) * di_ref[...]

    full = lambda shape: pl.BlockSpec(shape, lambda: tuple(0 for _ in shape))
    return pl.pallas_call(
        body,
        in_specs=[full((m_out, 128)), full((m_out // 4, 128)),
                  full((m_out, 128)), full((128, 128)),
                  full((4, 128, 128)), full((4, 128, 128))],
        out_specs=full((m_out, 128)),
        out_shape=_f32((m_out, 128)),
    )(a, c, dinv, Wb, EW, OW)


def _tc_combine(ap, s, dinv, bt, m_rows):
    def body(a_ref, s_ref, di_ref, b_ref, o_ref):
        acc = a_ref[0] + a_ref[1] + s_ref[...]
        o_ref[...] = jnp.maximum(di_ref[...] * acc + b_ref[...], 0.0)

    return pl.pallas_call(
        body,
        grid=(m_rows // _R4,),
        in_specs=[pl.BlockSpec((2, _R4, 128), lambda i: (0, i, 0)),
                  _rows(128), _rows(128), _const((1, 128))],
        out_specs=_rows(128),
        out_shape=_f32((m_rows, 128)),
    )(ap, s, dinv, bt)


def _tc_tail(ap, s, dinv, bt, fc2_Wb, fc2_bt):
    def body(a_ref, s_ref, di_ref, b_ref, fw_ref, fb_ref, o_ref):
        acc = a_ref[0] + a_ref[1] + s_ref[...]
        e = jnp.maximum(di_ref[...] * acc + b_ref[...], 0.0)
        o_ref[...] = lax.dot_general(
            fw_ref[...], e, (((0,), (1,)), ((), ())),
            preferred_element_type=jnp.float32, precision=lax.Precision.HIGHEST) + fb_ref[...]

    rt = 2944

    return pl.pallas_call(
        body,
        grid=(_M0 // rt,),
        in_specs=[pl.BlockSpec((2, rt, 128), lambda i: (0, i, 0)),
                  pl.BlockSpec((rt, 128), lambda i: (i, 0)),
                  pl.BlockSpec((rt, 128), lambda i: (i, 0)),
                  _const((1, 128)), _const((128, 12)), _const((12, 1))],
        out_specs=pl.BlockSpec((12, rt), lambda i: (0, i)),
        out_shape=_f32((12, _M0)),
    )(ap, s, dinv, bt, fc2_Wb, fc2_bt)


def _pad_flat(v, total, fill):
    return jnp.concatenate(
        [v.astype(jnp.int32),
         jnp.full((total - v.shape[0],), fill, jnp.int32)])


def _down_packed(a_pack, nx, ny, m_pad):
    n = nx * ny
    v = a_pack[:n // 4].reshape(nx, ny // 4, 128)[::2]
    d = jnp.concatenate([v[..., 0:32], v[..., 64:96]], axis=-1)
    d = d.reshape(n // 16, 128)
    return jnp.pad(d, ((0, m_pad - n // 16), (0, 0)))


def _blockdiag(W):
    return jnp.kron(jnp.eye(4, dtype=jnp.float32), W)


def _tile4(b):
    return jnp.tile(b.reshape(1, -1), (1, 4))


def kernel(x, edge_index_0, edge_index_1, edge_index_2, index_0, index_1,
           index_2, fc1_W, fc1_b, conv1_W, conv1_b, conv2_W, conv2_b,
           conv3_W, conv3_b, conv4_W, conv4_b, conv5_W, conv5_b,
           fc2_W, fc2_b):
    del index_0, index_1, index_2
    rowf0 = _pad_flat(edge_index_0[0], _EP0, 0)
    colf0 = _pad_flat(edge_index_0[1], _EP0, _N0)
    rowf1 = _pad_flat(edge_index_1[0], _EP1, 0)
    colf1 = _pad_flat(edge_index_1[1], _EP1, _N1)
    rowf2 = _pad_flat(edge_index_2[0], _EP2, 0)
    colf2 = _pad_flat(edge_index_2[1], _EP2, _N2)
    row3_0 = rowf0.reshape(32, _K0, _CPE)
    col3_0 = colf0.reshape(32, _K0, _CPE)
    row3_1 = rowf1.reshape(32, _K1, _CPE)
    col3_1 = colf1.reshape(32, _K1, _CPE)
    row3_2 = rowf2.reshape(32, _K2, _CPE)
    col3_2 = colf2.reshape(32, _K2, _CPE)
    col12 = jnp.concatenate([colf1, colf2 + _NP1]).reshape(32, _KD, _CPE)

    W1b = _blockdiag(conv1_W)
    W2b = _blockdiag(conv2_W)
    W3b = _blockdiag(conv3_W)
    W4b = _blockdiag(conv4_W)
    W5b = _blockdiag(conv5_W)
    fc1_Wb = _blockdiag(fc1_W)
    fc2_Wb = _blockdiag(fc2_W)
    ej = jnp.asarray(_EJ)
    oj = jnp.asarray(_OJ)
    EW4 = jnp.einsum("jab,bc->jac", ej, W4b)
    OW4 = jnp.einsum("jab,bc->jac", oj, W4b)
    EW5 = jnp.einsum("jab,bc->jac", ej, W5b)
    OW5 = jnp.einsum("jab,bc->jac", oj, W5b)

    dp0, dp12 = _sc_deg(col3_0, col12)
    dinv12 = _tc_dinv12(dp12.reshape(2, _N12 // 8, 128),
                        jnp.asarray(_XE), jnp.asarray(_XO))
    dinv1 = dinv12[:_M1]
    dinv2 = dinv12[_M1:]

    xp = jnp.pad(x.reshape(_N0 // 4, 16), ((0, _M0 - _N0 // 4), (0, 0)))
    s1, dinv0 = _tc_head(xp, dp0.reshape(2, _NP0 // 8, 128), fc1_Wb,
                         _tile4(fc1_b), W1b, jnp.asarray(_XE),
                         jnp.asarray(_XO))
    ap = _sc_conv(s1.reshape(_NP0, 32), row3_0, col3_0, _NP0)
    A = _tc_combine(ap.reshape(2, _M0, 128), s1, dinv0, _tile4(conv1_b), _M0)

    B0 = _down_packed(A, _NX, _NY, _M1)
    s2 = _tc_scale(B0, dinv1, W2b, _M1)
    bp = _sc_conv(s2.reshape(_NP1, 32), row3_1, col3_1, _NP1)
    B = _tc_combine(bp.reshape(2, _M1, 128), s2, dinv1, _tile4(conv2_b), _M1)

    C0 = _down_packed(B, _NX // 2, _NY // 2, _M2)
    s3 = _tc_scale(C0, dinv2, W3b, _M2)
    cp = _sc_conv(s3.reshape(_NP2, 32), row3_2, col3_2, _NP2)
    C = _tc_combine(cp.reshape(2, _M2, 128), s3, dinv2, _tile4(conv3_b), _M2)

    s4 = _tc_up_res_scale(B, C, dinv1, W4b, EW4, OW4, _M1, aligned=False)
    dpp = _sc_conv(s4.reshape(_NP1, 32), row3_1, col3_1, _NP1)
    D = _tc_combine(dpp.reshape(2, _M1, 128), s4, dinv1, _tile4(conv4_b), _M1)

    s5 = _tc_up_res_scale(A, D, dinv0, W5b, EW5, OW5, _M0, aligned=True,
                          off=_N1 // 8)
    ep = _sc_conv(s5.reshape(_NP0, 32), row3_0, col3_0, _NP0)
    outT = _tc_tail(ep.reshape(2, _M0, 128), s5, dinv0, _tile4(conv5_b),
                    fc2_Wb, jnp.tile(fc2_b, 4).reshape(12, 1))
    out = outT.reshape(4, 3, _M0).transpose(1, 2, 0).reshape(3, _NP0)
    return out[:, :_N0].T


# speedup vs baseline: 1.0734x; 1.0734x over previous
import functools

import jax
import jax.numpy as jnp
import numpy as np
from jax import lax
from jax.experimental import pallas as pl
from jax.experimental.pallas import tpu as pltpu
from jax.experimental.pallas import tpu_sc as plsc

_NX, _NY = 392, 120
_N0 = _NX * _NY
_N1 = (_NX // 2) * (_NY // 2)
_N2 = (_NX // 4) * (_NY // 4)
_NP0, _NP1, _NP2 = 47104, 11776, 2944
_N12 = _NP1 + _NP2
_M0, _M1, _M2 = _NP0 // 4, _NP1 // 4, _NP2 // 4
_CPE = 128
_EP0, _K0 = 188416, 46
_EP1, _K1 = 49152, 12
_EP2, _K2 = 16384, 4
_KD = _K1 + _K2
_R4 = 736


def _mesh():
    return plsc.VectorSubcoreMesh(core_axis_name="c", subcore_axis_name="s")


_SC_PARAMS = pltpu.CompilerParams(use_tc_tiling_on_sc=False)


def _sc_deg(col0, col12):
    rpt0 = _NP0 // 16
    rpt12 = _N12 // 16
    zr = 736
    nz0 = rpt0 // zr

    @functools.partial(
        pl.kernel,
        out_type=[jax.ShapeDtypeStruct((2, _NP0, 16), jnp.float32),
                  jax.ShapeDtypeStruct((2, _N12, 16), jnp.float32)],
        mesh=_mesh(),
        compiler_params=_SC_PARAMS,
        scratch_types=[
            pltpu.VMEM((_K0, _CPE), jnp.int32),
            pltpu.VMEM((_KD, _CPE), jnp.int32),
            pltpu.VMEM((_CPE, 16), jnp.float32),
            pltpu.VMEM((rpt12, 16), jnp.float32),
            pltpu.VMEM_SHARED((_NP0, 16), jnp.float32),
            pltpu.VMEM_SHARED((_N12, 16), jnp.float32),
            pltpu.SemaphoreType.DMA,
        ],
    )
    def kfn(col0_hbm, col12_hbm, out0_hbm, out12_hbm, coli0, coli12,
            ones16, zbuf16, dacc0, dacc12, sem):
        cid = lax.axis_index("c")
        sid = lax.axis_index("s")
        wid = cid * 16 + sid

        def fill16(i, _):
            zbuf16[i, :] = jnp.zeros((16,), jnp.float32)
            return _

        lax.fori_loop(0, rpt12, fill16, None)

        def fillo16(i, _):
            ones16[i, :] = jnp.ones((16,), jnp.float32)
            return _

        lax.fori_loop(0, _CPE, fillo16, None)

        def zc0(i, _):
            pltpu.sync_copy(zbuf16.at[pl.ds(0, zr)],
                            dacc0.at[pl.ds(sid * rpt0 + i * zr, zr)])
            return _

        lax.fori_loop(0, nz0, zc0, None)
        pltpu.sync_copy(zbuf16, dacc12.at[pl.ds(sid * rpt12, rpt12)])
        pltpu.sync_copy(col0_hbm.at[wid], coli0)
        pltpu.sync_copy(col12_hbm.at[wid], coli12)
        plsc.subcore_barrier()

        def edge0(j, _):
            pltpu.sync_copy(ones16, dacc0.at[coli0.at[j]], add=True)
            return _

        lax.fori_loop(0, _K0, edge0, None)

        def edge12(j, _):
            pltpu.sync_copy(ones16, dacc12.at[coli12.at[j]], add=True)
            return _

        lax.fori_loop(0, _KD, edge12, None)
        plsc.subcore_barrier()
        pltpu.sync_copy(dacc0.at[pl.ds(sid * rpt0, rpt0)],
                        out0_hbm.at[cid, pl.ds(sid * rpt0, rpt0)])
        pltpu.sync_copy(dacc12.at[pl.ds(sid * rpt12, rpt12)],
                        out12_hbm.at[cid, pl.ds(sid * rpt12, rpt12)])

    return kfn(col0, col12)


def _sc_conv(s, row3, col3, np_rows):
    k_ops = row3.shape[1]
    cpe = row3.shape[2]
    khalf = k_ops // 2
    rpt = np_rows // 16
    zr = 184
    nzero = rpt // zr

    @functools.partial(
        pl.kernel,
        out_type=jax.ShapeDtypeStruct((2, np_rows, 32), jnp.float32),
        mesh=_mesh(),
        compiler_params=_SC_PARAMS,
        scratch_types=[
            pltpu.VMEM((k_ops, cpe), jnp.int32),
            pltpu.VMEM((k_ops, cpe), jnp.int32),
            pltpu.VMEM((cpe, 32), jnp.float32),
            pltpu.VMEM((cpe, 32), jnp.float32),
            pltpu.VMEM((zr, 32), jnp.float32),
            pltpu.VMEM_SHARED((np_rows, 32), jnp.float32),
            pltpu.SemaphoreType.DMA,
            pltpu.SemaphoreType.DMA,
            pltpu.SemaphoreType.DMA,
            pltpu.SemaphoreType.DMA,
            pltpu.SemaphoreType.DMA,
        ],
    )
    def kfn(s_hbm, row_hbm, col_hbm, out_hbm, rowi, coli, buf0, buf1, zbuf,
            acc, semz, sem0, sem1, sems0, sems1):
        cid = lax.axis_index("c")
        sid = lax.axis_index("s")
        wid = cid * 16 + sid

        def fill(i, _):
            zbuf[i, pl.ds(0, 16)] = jnp.zeros((16,), jnp.float32)
            zbuf[i, pl.ds(16, 16)] = jnp.zeros((16,), jnp.float32)
            return _

        lax.fori_loop(0, zr, fill, None)

        pltpu.async_copy(row_hbm.at[wid], rowi, semz)
        pltpu.async_copy(col_hbm.at[wid], coli, semz)

        def zc(i, _):
            pltpu.sync_copy(zbuf, acc.at[pl.ds(sid * rpt + i * zr, zr)])
            return _

        lax.fori_loop(0, nzero, zc, None)
        pltpu.make_async_copy(row_hbm.at[wid], rowi, semz).wait()
        pltpu.make_async_copy(col_hbm.at[wid], coli, semz).wait()
        plsc.subcore_barrier()

        pltpu.async_copy(s_hbm.at[rowi.at[0]], buf0, sem0)

        def edge(jj, _):
            j = jj * 2

            @pl.when(jj > 0)
            def _wait_s1():
                pltpu.make_async_copy(buf1, acc.at[coli.at[j - 1]],
                                      sems1).wait()

            pltpu.async_copy(s_hbm.at[rowi.at[j + 1]], buf1, sem1)
            pltpu.make_async_copy(s_hbm.at[rowi.at[j]], buf0, sem0).wait()
            pltpu.async_copy(buf0, acc.at[coli.at[j]], sems0, add=True)

            @pl.when(j + 2 < k_ops)
            def _prefetch():
                pltpu.make_async_copy(buf0, acc.at[coli.at[j]], sems0).wait()
                pltpu.async_copy(s_hbm.at[rowi.at[j + 2]], buf0, sem0)

            pltpu.make_async_copy(s_hbm.at[rowi.at[j + 1]], buf1, sem1).wait()
            pltpu.async_copy(buf1, acc.at[coli.at[j + 1]], sems1, add=True)
            return _

        lax.fori_loop(0, khalf, edge, None)
        pltpu.make_async_copy(buf0, acc.at[coli.at[k_ops - 2]], sems0).wait()
        pltpu.make_async_copy(buf1, acc.at[coli.at[k_ops - 1]], sems1).wait()
        plsc.subcore_barrier()
        pltpu.sync_copy(acc.at[pl.ds(sid * rpt, rpt)],
                        out_hbm.at[cid, pl.ds(sid * rpt, rpt)])

    return kfn(s, row3, col3)


def _rows(c):
    return pl.BlockSpec((_R4, c), lambda i: (i, 0))


def _const(shape):
    return pl.BlockSpec(shape, lambda i: tuple(0 for _ in shape))


def _f32(shape):
    return jax.ShapeDtypeStruct(shape, jnp.float32)


_XE = np.zeros((128, 128), np.float32)
_XO = np.zeros((128, 128), np.float32)
for _i in range(4):
    for _c in range(32):
        _XE[16 * _i, 32 * _i + _c] = 1.0
        _XO[64 + 16 * _i, 32 * _i + _c] = 1.0


def _tc_dinv12(dp, xe, xo):
    m_rows = _N12 // 4
    qin = _R4 // 2

    def body(d_ref, xe_ref, xo_ref, o_ref):
        deg = d_ref[0] + d_ref[1]
        t = jnp.broadcast_to(deg[:, None, :], (qin, 2, 128)).reshape(_R4, 128)
        riota = lax.broadcasted_iota(jnp.int32, (_R4, 128), 0)
        val = jnp.where(
            riota % 2 == 0,
            jnp.dot(t, xe_ref[...], preferred_element_type=jnp.float32),
            jnp.dot(t, xo_ref[...], preferred_element_type=jnp.float32))
        o_ref[...] = lax.rsqrt(1.0 + val)

    return pl.pallas_call(
        body,
        grid=(m_rows // _R4,),
        in_specs=[pl.BlockSpec((2, qin, 128), lambda i: (0, i, 0)),
                  _const((128, 128)), _const((128, 128))],
        out_specs=_rows(128),
        out_shape=_f32((m_rows, 128)),
    )(dp, xe, xo)


def _tc_head(xp, dp0, fc1_Wb, fc1_bt, W1b, xe, xo):
    qin = _R4 // 2

    def body(x_ref, d_ref, fw_ref, fb_ref, w1_ref, xe_ref, xo_ref,
             s_ref, di_ref):
        deg = d_ref[0] + d_ref[1]
        t = jnp.broadcast_to(deg[:, None, :], (qin, 2, 128)).reshape(_R4, 128)
        riota = lax.broadcasted_iota(jnp.int32, (_R4, 128), 0)
        val = jnp.where(
            riota % 2 == 0,
            jnp.dot(t, xe_ref[...], preferred_element_type=jnp.float32),
            jnp.dot(t, xo_ref[...], preferred_element_type=jnp.float32))
        dinv = lax.rsqrt(1.0 + val)
        h = jnp.dot(x_ref[...], fw_ref[...], preferred_element_type=jnp.float32)
        h = jnp.maximum(h + fb_ref[...], 0.0)
        s_ref[...] = jnp.dot(h, w1_ref[...], preferred_element_type=jnp.float32) * dinv
        di_ref[...] = dinv

    return pl.pallas_call(
        body,
        grid=(_M0 // _R4,),
        in_specs=[_rows(16), pl.BlockSpec((2, qin, 128), lambda i: (0, i, 0)),
                  _const((16, 128)), _const((1, 128)), _const((128, 128)),
                  _const((128, 128)), _const((128, 128))],
        out_specs=[_rows(128), _rows(128)],
        out_shape=[_f32((_M0, 128)), _f32((_M0, 128))],
    )(xp, dp0, fc1_Wb, fc1_bt, W1b, xe, xo)


def _tc_scale(inp, dinv, Wb, m_rows):
    def body(x_ref, di_ref, w_ref, s_ref):
        s_ref[...] = jnp.dot(x_ref[...], w_ref[...], preferred_element_type=jnp.float32) * di_ref[...]

    return pl.pallas_call(
        body,
        grid=(m_rows // _R4,),
        in_specs=[_rows(128), _rows(128), _const((128, 128))],
        out_specs=_rows(128),
        out_shape=_f32((m_rows, 128)),
    )(inp, dinv, Wb)


_EJ = np.zeros((4, 128, 128), np.float32)
_OJ = np.zeros((4, 128, 128), np.float32)
for _j in range(4):
    for _b in (0, 1):
        for _u in (0, 1):
            for _cc in range(16):
                _EJ[_j, 32 * _j + 16 * _b + _cc, 64 * _u + 32 * _b + 2 * _cc] = 1.0
                _OJ[_j, 32 * _j + 16 * _b + _cc, 64 * _u + 32 * _b + 2 * _cc + 1] = 1.0


def _tc_up_res_scale(a, c, dinv, Wb, EW, OW, m_out, aligned, off=0):
    q = m_out // 8

    def body(a_ref, c_ref, di_ref, w_ref, ew_ref, ow_ref, s_ref):
        xa = c_ref[0:q, :]
        if aligned:
            xb = c_ref[off:off + q, :]
        else:
            xb = jnp.concatenate(
                [c_ref[q - 1:2 * q - 1, 64:128], c_ref[q:2 * q, 0:64]],
                axis=1)
        ys = [
            jnp.dot(xa, ew_ref[j], preferred_element_type=jnp.float32)
            + jnp.dot(xb, ow_ref[j], preferred_element_type=jnp.float32)
            for j in range(4)
        ]
        st = jnp.stack(ys, axis=1)
        st = jnp.broadcast_to(st[:, :, None, :], (q, 4, 2, 128))
        up = st.reshape(m_out, 128)
        base = jnp.dot(a_ref[...], w_ref[...], preferred_element_type=jnp.float32)
        s_ref[...] = (base + up) * di_ref[...]

    full = lambda shape: pl.BlockSpec(shape, lambda: tuple(0 for _ in shape))
    return pl.pallas_call(
        body,
        in_specs=[full((m_out, 128)), full((m_out // 4, 128)),
                  full((m_out, 128)), full((128, 128)),
                  full((4, 128, 128)), full((4, 128, 128))],
        out_specs=full((m_out, 128)),
        out_shape=_f32((m_out, 128)),
    )(a, c, dinv, Wb, EW, OW)


def _tc_combine(ap, s, dinv, bt, m_rows):
    def body(a_ref, s_ref, di_ref, b_ref, o_ref):
        acc = a_ref[0] + a_ref[1] + s_ref[...]
        o_ref[...] = jnp.maximum(di_ref[...] * acc + b_ref[...], 0.0)

    return pl.pallas_call(
        body,
        grid=(m_rows // _R4,),
        in_specs=[pl.BlockSpec((2, _R4, 128), lambda i: (0, i, 0)),
                  _rows(128), _rows(128), _const((1, 128))],
        out_specs=_rows(128),
        out_shape=_f32((m_rows, 128)),
    )(ap, s, dinv, bt)


def _tc_tail(ap, s, dinv, bt, fc2_Wb, fc2_bt):
    def body(a_ref, s_ref, di_ref, b_ref, fw_ref, fb_ref, o_ref):
        acc = a_ref[0] + a_ref[1] + s_ref[...]
        e = jnp.maximum(di_ref[...] * acc + b_ref[...], 0.0)
        o_ref[...] = lax.dot_general(
            fw_ref[...], e, (((0,), (1,)), ((), ())),
            preferred_element_type=jnp.float32) + fb_ref[...]

    rt = 2944

    return pl.pallas_call(
        body,
        grid=(_M0 // rt,),
        in_specs=[pl.BlockSpec((2, rt, 128), lambda i: (0, i, 0)),
                  pl.BlockSpec((rt, 128), lambda i: (i, 0)),
                  pl.BlockSpec((rt, 128), lambda i: (i, 0)),
                  _const((1, 128)), _const((128, 12)), _const((12, 1))],
        out_specs=pl.BlockSpec((12, rt), lambda i: (0, i)),
        out_shape=_f32((12, _M0)),
    )(ap, s, dinv, bt, fc2_Wb, fc2_bt)


def _pad_flat(v, total, fill):
    return jnp.concatenate(
        [v.astype(jnp.int32),
         jnp.full((total - v.shape[0],), fill, jnp.int32)])


def _down_packed(a_pack, nx, ny, m_pad):
    n = nx * ny
    v = a_pack[:n // 4].reshape(nx, ny // 4, 128)[::2]
    d = jnp.concatenate([v[..., 0:32], v[..., 64:96]], axis=-1)
    d = d.reshape(n // 16, 128)
    return jnp.pad(d, ((0, m_pad - n // 16), (0, 0)))


def _blockdiag(W):
    return jnp.kron(jnp.eye(4, dtype=jnp.float32), W)


def _tile4(b):
    return jnp.tile(b.reshape(1, -1), (1, 4))


def kernel(x, edge_index_0, edge_index_1, edge_index_2, index_0, index_1,
           index_2, fc1_W, fc1_b, conv1_W, conv1_b, conv2_W, conv2_b,
           conv3_W, conv3_b, conv4_W, conv4_b, conv5_W, conv5_b,
           fc2_W, fc2_b):
    del index_0, index_1, index_2
    rowf0 = _pad_flat(edge_index_0[0], _EP0, 0)
    colf0 = _pad_flat(edge_index_0[1], _EP0, _N0)
    rowf1 = _pad_flat(edge_index_1[0], _EP1, 0)
    colf1 = _pad_flat(edge_index_1[1], _EP1, _N1)
    rowf2 = _pad_flat(edge_index_2[0], _EP2, 0)
    colf2 = _pad_flat(edge_index_2[1], _EP2, _N2)
    row3_0 = rowf0.reshape(32, _K0, _CPE)
    col3_0 = colf0.reshape(32, _K0, _CPE)
    row3_1 = rowf1.reshape(32, _K1, _CPE)
    col3_1 = colf1.reshape(32, _K1, _CPE)
    row3_2 = rowf2.reshape(32, _K2, _CPE)
    col3_2 = colf2.reshape(32, _K2, _CPE)
    col12 = jnp.concatenate([colf1, colf2 + _NP1]).reshape(32, _KD, _CPE)

    W1b = _blockdiag(conv1_W)
    W2b = _blockdiag(conv2_W)
    W3b = _blockdiag(conv3_W)
    W4b = _blockdiag(conv4_W)
    W5b = _blockdiag(conv5_W)
    fc1_Wb = _blockdiag(fc1_W)
    fc2_Wb = _blockdiag(fc2_W)
    ej = jnp.asarray(_EJ)
    oj = jnp.asarray(_OJ)
    EW4 = jnp.einsum("jab,bc->jac", ej, W4b, precision=lax.Precision.HIGHEST)
    OW4 = jnp.einsum("jab,bc->jac", oj, W4b, precision=lax.Precision.HIGHEST)
    EW5 = jnp.einsum("jab,bc->jac", ej, W5b, precision=lax.Precision.HIGHEST)
    OW5 = jnp.einsum("jab,bc->jac", oj, W5b, precision=lax.Precision.HIGHEST)

    dp0, dp12 = _sc_deg(col3_0, col12)
    dinv12 = _tc_dinv12(dp12.reshape(2, _N12 // 8, 128),
                        jnp.asarray(_XE), jnp.asarray(_XO))
    dinv1 = dinv12[:_M1]
    dinv2 = dinv12[_M1:]

    xp = jnp.pad(x.reshape(_N0 // 4, 16), ((0, _M0 - _N0 // 4), (0, 0)))
    s1, dinv0 = _tc_head(xp, dp0.reshape(2, _NP0 // 8, 128), fc1_Wb,
                         _tile4(fc1_b), W1b, jnp.asarray(_XE),
                         jnp.asarray(_XO))
    ap = _sc_conv(s1.reshape(_NP0, 32), row3_0, col3_0, _NP0)
    A = _tc_combine(ap.reshape(2, _M0, 128), s1, dinv0, _tile4(conv1_b), _M0)

    B0 = _down_packed(A, _NX, _NY, _M1)
    s2 = _tc_scale(B0, dinv1, W2b, _M1)
    bp = _sc_conv(s2.reshape(_NP1, 32), row3_1, col3_1, _NP1)
    B = _tc_combine(bp.reshape(2, _M1, 128), s2, dinv1, _tile4(conv2_b), _M1)

    C0 = _down_packed(B, _NX // 2, _NY // 2, _M2)
    s3 = _tc_scale(C0, dinv2, W3b, _M2)
    cp = _sc_conv(s3.reshape(_NP2, 32), row3_2, col3_2, _NP2)
    C = _tc_combine(cp.reshape(2, _M2, 128), s3, dinv2, _tile4(conv3_b), _M2)

    s4 = _tc_up_res_scale(B, C, dinv1, W4b, EW4, OW4, _M1, aligned=False)
    dpp = _sc_conv(s4.reshape(_NP1, 32), row3_1, col3_1, _NP1)
    D = _tc_combine(dpp.reshape(2, _M1, 128), s4, dinv1, _tile4(conv4_b), _M1)

    s5 = _tc_up_res_scale(A, D, dinv0, W5b, EW5, OW5, _M0, aligned=True,
                          off=_N1 // 8)
    ep = _sc_conv(s5.reshape(_NP0, 32), row3_0, col3_0, _NP0)
    outT = _tc_tail(ep.reshape(2, _M0, 128), s5, dinv0, _tile4(conv5_b),
                    fc2_Wb, jnp.tile(fc2_b, 4).reshape(12, 1))
    out = outT.reshape(4, 3, _M0).transpose(1, 2, 0).reshape(3, _NP0)
    return out[:, :_N0].T


# speedup vs baseline: 1.0741x; 1.0006x over previous
import functools

import jax
import jax.numpy as jnp
import numpy as np
from jax import lax
from jax.experimental import pallas as pl
from jax.experimental.pallas import tpu as pltpu
from jax.experimental.pallas import tpu_sc as plsc

_NX, _NY = 392, 120
_N0 = _NX * _NY
_N1 = (_NX // 2) * (_NY // 2)
_N2 = (_NX // 4) * (_NY // 4)
_NP0, _NP1, _NP2 = 47104, 11776, 2944
_N12 = _NP1 + _NP2
_M0, _M1, _M2 = _NP0 // 4, _NP1 // 4, _NP2 // 4
_CPE = 128
_EP0, _K0 = 188416, 46
_EP1, _K1 = 49152, 12
_EP2, _K2 = 16384, 4
_KD = _K1 + _K2
_R4 = 736


def _mesh():
    return plsc.VectorSubcoreMesh(core_axis_name="c", subcore_axis_name="s")


_SC_PARAMS = pltpu.CompilerParams(use_tc_tiling_on_sc=False,
                                  skip_device_barrier=True)


def _sc_deg(col0, col12):
    rpt0 = _NP0 // 16
    rpt12 = _N12 // 16
    zr = 736
    nz0 = rpt0 // zr

    @functools.partial(
        pl.kernel,
        out_type=[jax.ShapeDtypeStruct((2, _NP0, 16), jnp.float32),
                  jax.ShapeDtypeStruct((2, _N12, 16), jnp.float32)],
        mesh=_mesh(),
        compiler_params=_SC_PARAMS,
        scratch_types=[
            pltpu.VMEM((_K0, _CPE), jnp.int32),
            pltpu.VMEM((_KD, _CPE), jnp.int32),
            pltpu.VMEM((_CPE, 16), jnp.float32),
            pltpu.VMEM((rpt12, 16), jnp.float32),
            pltpu.VMEM_SHARED((_NP0, 16), jnp.float32),
            pltpu.VMEM_SHARED((_N12, 16), jnp.float32),
            pltpu.SemaphoreType.DMA,
        ],
    )
    def kfn(col0_hbm, col12_hbm, out0_hbm, out12_hbm, coli0, coli12,
            ones16, zbuf16, dacc0, dacc12, sem):
        cid = lax.axis_index("c")
        sid = lax.axis_index("s")
        wid = cid * 16 + sid

        def fill16(i, _):
            zbuf16[i, :] = jnp.zeros((16,), jnp.float32)
            return _

        lax.fori_loop(0, rpt12, fill16, None)

        def fillo16(i, _):
            ones16[i, :] = jnp.ones((16,), jnp.float32)
            return _

        lax.fori_loop(0, _CPE, fillo16, None)

        def zc0(i, _):
            pltpu.sync_copy(zbuf16.at[pl.ds(0, zr)],
                            dacc0.at[pl.ds(sid * rpt0 + i * zr, zr)])
            return _

        lax.fori_loop(0, nz0, zc0, None)
        pltpu.sync_copy(zbuf16, dacc12.at[pl.ds(sid * rpt12, rpt12)])
        pltpu.sync_copy(col0_hbm.at[wid], coli0)
        pltpu.sync_copy(col12_hbm.at[wid], coli12)
        plsc.subcore_barrier()

        def edge0(j, _):
            pltpu.sync_copy(ones16, dacc0.at[coli0.at[j]], add=True)
            return _

        lax.fori_loop(0, _K0, edge0, None)

        def edge12(j, _):
            pltpu.sync_copy(ones16, dacc12.at[coli12.at[j]], add=True)
            return _

        lax.fori_loop(0, _KD, edge12, None)
        plsc.subcore_barrier()
        pltpu.sync_copy(dacc0.at[pl.ds(sid * rpt0, rpt0)],
                        out0_hbm.at[cid, pl.ds(sid * rpt0, rpt0)])
        pltpu.sync_copy(dacc12.at[pl.ds(sid * rpt12, rpt12)],
                        out12_hbm.at[cid, pl.ds(sid * rpt12, rpt12)])

    return kfn(col0, col12)


def _sc_conv(s, row3, col3, np_rows):
    k_ops = row3.shape[1]
    cpe = row3.shape[2]
    khalf = k_ops // 2
    rpt = np_rows // 16
    zr = 184
    nzero = rpt // zr

    @functools.partial(
        pl.kernel,
        out_type=jax.ShapeDtypeStruct((2, np_rows, 32), jnp.float32),
        mesh=_mesh(),
        compiler_params=_SC_PARAMS,
        scratch_types=[
            pltpu.VMEM((k_ops, cpe), jnp.int32),
            pltpu.VMEM((k_ops, cpe), jnp.int32),
            pltpu.VMEM((cpe, 32), jnp.float32),
            pltpu.VMEM((cpe, 32), jnp.float32),
            pltpu.VMEM((zr, 32), jnp.float32),
            pltpu.VMEM_SHARED((np_rows, 32), jnp.float32),
            pltpu.SemaphoreType.DMA,
            pltpu.SemaphoreType.DMA,
            pltpu.SemaphoreType.DMA,
            pltpu.SemaphoreType.DMA,
            pltpu.SemaphoreType.DMA,
        ],
    )
    def kfn(s_hbm, row_hbm, col_hbm, out_hbm, rowi, coli, buf0, buf1, zbuf,
            acc, semz, sem0, sem1, sems0, sems1):
        cid = lax.axis_index("c")
        sid = lax.axis_index("s")
        wid = cid * 16 + sid

        def fill(i, _):
            zbuf[i, pl.ds(0, 16)] = jnp.zeros((16,), jnp.float32)
            zbuf[i, pl.ds(16, 16)] = jnp.zeros((16,), jnp.float32)
            return _

        lax.fori_loop(0, zr, fill, None)

        pltpu.async_copy(row_hbm.at[wid], rowi, semz)
        pltpu.async_copy(col_hbm.at[wid], coli, semz)

        def zc(i, _):
            pltpu.sync_copy(zbuf, acc.at[pl.ds(sid * rpt + i * zr, zr)])
            return _

        lax.fori_loop(0, nzero, zc, None)
        pltpu.make_async_copy(row_hbm.at[wid], rowi, semz).wait()
        pltpu.make_async_copy(col_hbm.at[wid], coli, semz).wait()
        plsc.subcore_barrier()

        pltpu.async_copy(s_hbm.at[rowi.at[0]], buf0, sem0)

        def edge(jj, _):
            j = jj * 2

            @pl.when(jj > 0)
            def _wait_s1():
                pltpu.make_async_copy(buf1, acc.at[coli.at[j - 1]],
                                      sems1).wait()

            pltpu.async_copy(s_hbm.at[rowi.at[j + 1]], buf1, sem1)
            pltpu.make_async_copy(s_hbm.at[rowi.at[j]], buf0, sem0).wait()
            pltpu.async_copy(buf0, acc.at[coli.at[j]], sems0, add=True)

            @pl.when(j + 2 < k_ops)
            def _prefetch():
                pltpu.make_async_copy(buf0, acc.at[coli.at[j]], sems0).wait()
                pltpu.async_copy(s_hbm.at[rowi.at[j + 2]], buf0, sem0)

            pltpu.make_async_copy(s_hbm.at[rowi.at[j + 1]], buf1, sem1).wait()
            pltpu.async_copy(buf1, acc.at[coli.at[j + 1]], sems1, add=True)
            return _

        lax.fori_loop(0, khalf, edge, None)
        pltpu.make_async_copy(buf0, acc.at[coli.at[k_ops - 2]], sems0).wait()
        pltpu.make_async_copy(buf1, acc.at[coli.at[k_ops - 1]], sems1).wait()
        plsc.subcore_barrier()
        pltpu.sync_copy(acc.at[pl.ds(sid * rpt, rpt)],
                        out_hbm.at[cid, pl.ds(sid * rpt, rpt)])

    return kfn(s, row3, col3)


def _rows(c):
    return pl.BlockSpec((_R4, c), lambda i: (i, 0))


def _const(shape):
    return pl.BlockSpec(shape, lambda i: tuple(0 for _ in shape))


def _f32(shape):
    return jax.ShapeDtypeStruct(shape, jnp.float32)


_XE = np.zeros((128, 128), np.float32)
_XO = np.zeros((128, 128), np.float32)
for _i in range(4):
    for _c in range(32):
        _XE[16 * _i, 32 * _i + _c] = 1.0
        _XO[64 + 16 * _i, 32 * _i + _c] = 1.0


def _tc_dinv12(dp, xe, xo):
    m_rows = _N12 // 4
    qin = _R4 // 2

    def body(d_ref, xe_ref, xo_ref, o_ref):
        deg = d_ref[0] + d_ref[1]
        t = jnp.broadcast_to(deg[:, None, :], (qin, 2, 128)).reshape(_R4, 128)
        riota = lax.broadcasted_iota(jnp.int32, (_R4, 128), 0)
        val = jnp.where(
            riota % 2 == 0,
            jnp.dot(t, xe_ref[...], preferred_element_type=jnp.float32),
            jnp.dot(t, xo_ref[...], preferred_element_type=jnp.float32))
        o_ref[...] = lax.rsqrt(1.0 + val)

    return pl.pallas_call(
        body,
        grid=(m_rows // _R4,),
        in_specs=[pl.BlockSpec((2, qin, 128), lambda i: (0, i, 0)),
                  _const((128, 128)), _const((128, 128))],
        out_specs=_rows(128),
        out_shape=_f32((m_rows, 128)),
    )(dp, xe, xo)


def _tc_head(xp, dp0, fc1_Wb, fc1_bt, W1b, xe, xo):
    qin = _R4 // 2

    def body(x_ref, d_ref, fw_ref, fb_ref, w1_ref, xe_ref, xo_ref,
             s_ref, di_ref):
        deg = d_ref[0] + d_ref[1]
        t = jnp.broadcast_to(deg[:, None, :], (qin, 2, 128)).reshape(_R4, 128)
        riota = lax.broadcasted_iota(jnp.int32, (_R4, 128), 0)
        val = jnp.where(
            riota % 2 == 0,
            jnp.dot(t, xe_ref[...], preferred_element_type=jnp.float32),
            jnp.dot(t, xo_ref[...], preferred_element_type=jnp.float32))
        dinv = lax.rsqrt(1.0 + val)
        h = jnp.dot(x_ref[...], fw_ref[...], preferred_element_type=jnp.float32)
        h = jnp.maximum(h + fb_ref[...], 0.0)
        s_ref[...] = jnp.dot(h, w1_ref[...], preferred_element_type=jnp.float32) * dinv
        di_ref[...] = dinv

    return pl.pallas_call(
        body,
        grid=(_M0 // _R4,),
        in_specs=[_rows(16), pl.BlockSpec((2, qin, 128), lambda i: (0, i, 0)),
                  _const((16, 128)), _const((1, 128)), _const((128, 128)),
                  _const((128, 128)), _const((128, 128))],
        out_specs=[_rows(128), _rows(128)],
        out_shape=[_f32((_M0, 128)), _f32((_M0, 128))],
    )(xp, dp0, fc1_Wb, fc1_bt, W1b, xe, xo)


def _tc_scale(inp, dinv, Wb, m_rows):
    def body(x_ref, di_ref, w_ref, s_ref):
        s_ref[...] = jnp.dot(x_ref[...], w_ref[...], preferred_element_type=jnp.float32) * di_ref[...]

    return pl.pallas_call(
        body,
        grid=(m_rows // _R4,),
        in_specs=[_rows(128), _rows(128), _const((128, 128))],
        out_specs=_rows(128),
        out_shape=_f32((m_rows, 128)),
    )(inp, dinv, Wb)


_EJ = np.zeros((4, 128, 128), np.float32)
_OJ = np.zeros((4, 128, 128), np.float32)
for _j in range(4):
    for _b in (0, 1):
        for _u in (0, 1):
            for _cc in range(16):
                _EJ[_j, 32 * _j + 16 * _b + _cc, 64 * _u + 32 * _b + 2 * _cc] = 1.0
                _OJ[_j, 32 * _j + 16 * _b + _cc, 64 * _u + 32 * _b + 2 * _cc + 1] = 1.0


def _tc_up_res_scale(a, c, dinv, Wb, EW, OW, m_out, aligned, off=0):
    q = m_out // 8

    def body(a_ref, c_ref, di_ref, w_ref, ew_ref, ow_ref, s_ref):
        xa = c_ref[0:q, :]
        if aligned:
            xb = c_ref[off:off + q, :]
        else:
            xb = jnp.concatenate(
                [c_ref[q - 1:2 * q - 1, 64:128], c_ref[q:2 * q, 0:64]],
                axis=1)
        ys = [
            jnp.dot(xa, ew_ref[j], preferred_element_type=jnp.float32)
            + jnp.dot(xb, ow_ref[j], preferred_element_type=jnp.float32)
            for j in range(4)
        ]
        st = jnp.stack(ys, axis=1)
        st = jnp.broadcast_to(st[:, :, None, :], (q, 4, 2, 128))
        up = st.reshape(m_out, 128)
        base = jnp.dot(a_ref[...], w_ref[...], preferred_element_type=jnp.float32)
        s_ref[...] = (base + up) * di_ref[...]

    full = lambda shape: pl.BlockSpec(shape, lambda: tuple(0 for _ in shape))
    return pl.pallas_call(
        body,
        in_specs=[full((m_out, 128)), full((m_out // 4, 128)),
                  full((m_out, 128)), full((128, 128)),
                  full((4, 128, 128)), full((4, 128, 128))],
        out_specs=full((m_out, 128)),
        out_shape=_f32((m_out, 128)),
    )(a, c, dinv, Wb, EW, OW)


def _tc_combine(ap, s, dinv, bt, m_rows):
    def body(a_ref, s_ref, di_ref, b_ref, o_ref):
        acc = a_ref[0] + a_ref[1] + s_ref[...]
        o_ref[...] = jnp.maximum(di_ref[...] * acc + b_ref[...], 0.0)

    return pl.pallas_call(
        body,
        grid=(m_rows // _R4,),
        in_specs=[pl.BlockSpec((2, _R4, 128), lambda i: (0, i, 0)),
                  _rows(128), _rows(128), _const((1, 128))],
        out_specs=_rows(128),
        out_shape=_f32((m_rows, 128)),
    )(ap, s, dinv, bt)


def _tc_tail(ap, s, dinv, bt, fc2_Wb, fc2_bt):
    def body(a_ref, s_ref, di_ref, b_ref, fw_ref, fb_ref, o_ref):
        acc = a_ref[0] + a_ref[1] + s_ref[...]
        e = jnp.maximum(di_ref[...] * acc + b_ref[...], 0.0)
        o_ref[...] = lax.dot_general(
            fw_ref[...], e, (((0,), (1,)), ((), ())),
            preferred_element_type=jnp.float32) + fb_ref[...]

    rt = 2944

    return pl.pallas_call(
        body,
        grid=(_M0 // rt,),
        in_specs=[pl.BlockSpec((2, rt, 128), lambda i: (0, i, 0)),
                  pl.BlockSpec((rt, 128), lambda i: (i, 0)),
                  pl.BlockSpec((rt, 128), lambda i: (i, 0)),
                  _const((1, 128)), _const((128, 12)), _const((12, 1))],
        out_specs=pl.BlockSpec((12, rt), lambda i: (0, i)),
        out_shape=_f32((12, _M0)),
    )(ap, s, dinv, bt, fc2_Wb, fc2_bt)


def _pad_flat(v, total, fill):
    return jnp.concatenate(
        [v.astype(jnp.int32),
         jnp.full((total - v.shape[0],), fill, jnp.int32)])


def _down_packed(a_pack, nx, ny, m_pad):
    n = nx * ny
    v = a_pack[:n // 4].reshape(nx, ny // 4, 128)[::2]
    d = jnp.concatenate([v[..., 0:32], v[..., 64:96]], axis=-1)
    d = d.reshape(n // 16, 128)
    return jnp.pad(d, ((0, m_pad - n // 16), (0, 0)))


def _blockdiag(W):
    return jnp.kron(jnp.eye(4, dtype=jnp.float32), W)


def _tile4(b):
    return jnp.tile(b.reshape(1, -1), (1, 4))


def kernel(x, edge_index_0, edge_index_1, edge_index_2, index_0, index_1,
           index_2, fc1_W, fc1_b, conv1_W, conv1_b, conv2_W, conv2_b,
           conv3_W, conv3_b, conv4_W, conv4_b, conv5_W, conv5_b,
           fc2_W, fc2_b):
    del index_0, index_1, index_2
    rowf0 = _pad_flat(edge_index_0[0], _EP0, 0)
    colf0 = _pad_flat(edge_index_0[1], _EP0, _N0)
    rowf1 = _pad_flat(edge_index_1[0], _EP1, 0)
    colf1 = _pad_flat(edge_index_1[1], _EP1, _N1)
    rowf2 = _pad_flat(edge_index_2[0], _EP2, 0)
    colf2 = _pad_flat(edge_index_2[1], _EP2, _N2)
    row3_0 = rowf0.reshape(32, _K0, _CPE)
    col3_0 = colf0.reshape(32, _K0, _CPE)
    row3_1 = rowf1.reshape(32, _K1, _CPE)
    col3_1 = colf1.reshape(32, _K1, _CPE)
    row3_2 = rowf2.reshape(32, _K2, _CPE)
    col3_2 = colf2.reshape(32, _K2, _CPE)
    col12 = jnp.concatenate([colf1, colf2 + _NP1]).reshape(32, _KD, _CPE)

    W1b = _blockdiag(conv1_W)
    W2b = _blockdiag(conv2_W)
    W3b = _blockdiag(conv3_W)
    W4b = _blockdiag(conv4_W)
    W5b = _blockdiag(conv5_W)
    fc1_Wb = _blockdiag(fc1_W)
    fc2_Wb = _blockdiag(fc2_W)
    ej = jnp.asarray(_EJ)
    oj = jnp.asarray(_OJ)
    EW4 = jnp.einsum("jab,bc->jac", ej, W4b, precision=lax.Precision.HIGHEST)
    OW4 = jnp.einsum("jab,bc->jac", oj, W4b, precision=lax.Precision.HIGHEST)
    EW5 = jnp.einsum("jab,bc->jac", ej, W5b, precision=lax.Precision.HIGHEST)
    OW5 = jnp.einsum("jab,bc->jac", oj, W5b, precision=lax.Precision.HIGHEST)

    dp0, dp12 = _sc_deg(col3_0, col12)
    dinv12 = _tc_dinv12(dp12.reshape(2, _N12 // 8, 128),
                        jnp.asarray(_XE), jnp.asarray(_XO))
    dinv1 = dinv12[:_M1]
    dinv2 = dinv12[_M1:]

    xp = jnp.pad(x.reshape(_N0 // 4, 16), ((0, _M0 - _N0 // 4), (0, 0)))
    s1, dinv0 = _tc_head(xp, dp0.reshape(2, _NP0 // 8, 128), fc1_Wb,
                         _tile4(fc1_b), W1b, jnp.asarray(_XE),
                         jnp.asarray(_XO))
    ap = _sc_conv(s1.reshape(_NP0, 32), row3_0, col3_0, _NP0)
    A = _tc_combine(ap.reshape(2, _M0, 128), s1, dinv0, _tile4(conv1_b), _M0)

    B0 = _down_packed(A, _NX, _NY, _M1)
    s2 = _tc_scale(B0, dinv1, W2b, _M1)
    bp = _sc_conv(s2.reshape(_NP1, 32), row3_1, col3_1, _NP1)
    B = _tc_combine(bp.reshape(2, _M1, 128), s2, dinv1, _tile4(conv2_b), _M1)

    C0 = _down_packed(B, _NX // 2, _NY // 2, _M2)
    s3 = _tc_scale(C0, dinv2, W3b, _M2)
    cp = _sc_conv(s3.reshape(_NP2, 32), row3_2, col3_2, _NP2)
    C = _tc_combine(cp.reshape(2, _M2, 128), s3, dinv2, _tile4(conv3_b), _M2)

    s4 = _tc_up_res_scale(B, C, dinv1, W4b, EW4, OW4, _M1, aligned=False)
    dpp = _sc_conv(s4.reshape(_NP1, 32), row3_1, col3_1, _NP1)
    D = _tc_combine(dpp.reshape(2, _M1, 128), s4, dinv1, _tile4(conv4_b), _M1)

    s5 = _tc_up_res_scale(A, D, dinv0, W5b, EW5, OW5, _M0, aligned=True,
                          off=_N1 // 8)
    ep = _sc_conv(s5.reshape(_NP0, 32), row3_0, col3_0, _NP0)
    outT = _tc_tail(ep.reshape(2, _M0, 128), s5, dinv0, _tile4(conv5_b),
                    fc2_Wb, jnp.tile(fc2_b, 4).reshape(12, 1))
    out = outT.reshape(4, 3, _M0).transpose(1, 2, 0).reshape(3, _NP0)
    return out[:, :_N0].T


# speedup vs baseline: 1.0742x; 1.0002x over previous
import functools

import jax
import jax.numpy as jnp
import numpy as np
from jax import lax
from jax.experimental import pallas as pl
from jax.experimental.pallas import tpu as pltpu
from jax.experimental.pallas import tpu_sc as plsc

_NX, _NY = 392, 120
_N0 = _NX * _NY
_N1 = (_NX // 2) * (_NY // 2)
_N2 = (_NX // 4) * (_NY // 4)
_NP0, _NP1, _NP2 = 47104, 11776, 2944
_N12 = _NP1 + _NP2
_M0, _M1, _M2 = _NP0 // 4, _NP1 // 4, _NP2 // 4
_CPE = 128
_EP0, _K0 = 188416, 46
_EP1, _K1 = 49152, 12
_EP2, _K2 = 16384, 4
_KD = _K1 + _K2
_R4 = 736


def _mesh():
    return plsc.VectorSubcoreMesh(core_axis_name="c", subcore_axis_name="s")


_SC_PARAMS = pltpu.CompilerParams(use_tc_tiling_on_sc=False)


def _sc_deg(col0, col12):
    rpt0 = _NP0 // 16
    rpt12 = _N12 // 16
    zr = 736
    nz0 = rpt0 // zr

    @functools.partial(
        pl.kernel,
        out_type=[jax.ShapeDtypeStruct((2, _NP0, 16), jnp.float32),
                  jax.ShapeDtypeStruct((2, _N12, 16), jnp.float32)],
        mesh=_mesh(),
        compiler_params=_SC_PARAMS,
        scratch_types=[
            pltpu.VMEM((_K0, _CPE), jnp.int32),
            pltpu.VMEM((_KD, _CPE), jnp.int32),
            pltpu.VMEM((_CPE, 16), jnp.float32),
            pltpu.VMEM((rpt12, 16), jnp.float32),
            pltpu.VMEM_SHARED((_NP0, 16), jnp.float32),
            pltpu.VMEM_SHARED((_N12, 16), jnp.float32),
            pltpu.SemaphoreType.DMA,
        ],
    )
    def kfn(col0_hbm, col12_hbm, out0_hbm, out12_hbm, coli0, coli12,
            ones16, zbuf16, dacc0, dacc12, sem):
        cid = lax.axis_index("c")
        sid = lax.axis_index("s")
        wid = cid * 16 + sid

        def fill16(i, _):
            zbuf16[i, :] = jnp.zeros((16,), jnp.float32)
            return _

        lax.fori_loop(0, rpt12, fill16, None)

        def fillo16(i, _):
            ones16[i, :] = jnp.ones((16,), jnp.float32)
            return _

        lax.fori_loop(0, _CPE, fillo16, None)

        def zc0(i, _):
            pltpu.sync_copy(zbuf16.at[pl.ds(0, zr)],
                            dacc0.at[pl.ds(sid * rpt0 + i * zr, zr)])
            return _

        lax.fori_loop(0, nz0, zc0, None)
        pltpu.sync_copy(zbuf16, dacc12.at[pl.ds(sid * rpt12, rpt12)])
        pltpu.sync_copy(col0_hbm.at[wid], coli0)
        pltpu.sync_copy(col12_hbm.at[wid], coli12)
        plsc.subcore_barrier()

        def edge0(j, _):
            pltpu.sync_copy(ones16, dacc0.at[coli0.at[j]], add=True)
            return _

        lax.fori_loop(0, _K0, edge0, None)

        def edge12(j, _):
            pltpu.sync_copy(ones16, dacc12.at[coli12.at[j]], add=True)
            return _

        lax.fori_loop(0, _KD, edge12, None)
        plsc.subcore_barrier()
        pltpu.sync_copy(dacc0.at[pl.ds(sid * rpt0, rpt0)],
                        out0_hbm.at[cid, pl.ds(sid * rpt0, rpt0)])
        pltpu.sync_copy(dacc12.at[pl.ds(sid * rpt12, rpt12)],
                        out12_hbm.at[cid, pl.ds(sid * rpt12, rpt12)])

    return kfn(col0, col12)


def _sc_conv(s, row3, col3, np_rows):
    k_ops = row3.shape[1]
    cpe = row3.shape[2]
    khalf = k_ops // 2
    rpt = np_rows // 16
    zr = 184
    nzero = rpt // zr

    @functools.partial(
        pl.kernel,
        out_type=jax.ShapeDtypeStruct((2, np_rows, 32), jnp.float32),
        mesh=_mesh(),
        compiler_params=_SC_PARAMS,
        scratch_types=[
            pltpu.VMEM((k_ops, cpe), jnp.int32),
            pltpu.VMEM((k_ops, cpe), jnp.int32),
            pltpu.VMEM((cpe, 32), jnp.float32),
            pltpu.VMEM((cpe, 32), jnp.float32),
            pltpu.VMEM((zr, 32), jnp.float32),
            pltpu.VMEM_SHARED((np_rows, 32), jnp.float32),
            pltpu.SemaphoreType.DMA,
            pltpu.SemaphoreType.DMA,
            pltpu.SemaphoreType.DMA,
            pltpu.SemaphoreType.DMA,
            pltpu.SemaphoreType.DMA,
        ],
    )
    def kfn(s_hbm, row_hbm, col_hbm, out_hbm, rowi, coli, buf0, buf1, zbuf,
            acc, semz, sem0, sem1, sems0, sems1):
        cid = lax.axis_index("c")
        sid = lax.axis_index("s")
        wid = cid * 16 + sid

        def fill(i, _):
            zbuf[i, pl.ds(0, 16)] = jnp.zeros((16,), jnp.float32)
            zbuf[i, pl.ds(16, 16)] = jnp.zeros((16,), jnp.float32)
            return _

        lax.fori_loop(0, zr, fill, None)

        pltpu.async_copy(row_hbm.at[wid], rowi, semz)
        pltpu.async_copy(col_hbm.at[wid], coli, semz)

        def zc(i, _):
            pltpu.sync_copy(zbuf, acc.at[pl.ds(sid * rpt + i * zr, zr)])
            return _

        lax.fori_loop(0, nzero, zc, None)
        pltpu.make_async_copy(row_hbm.at[wid], rowi, semz).wait()
        pltpu.make_async_copy(col_hbm.at[wid], coli, semz).wait()
        plsc.subcore_barrier()

        pltpu.async_copy(s_hbm.at[rowi.at[0]], buf0, sem0)

        def edge(jj, _):
            j = jj * 2

            @pl.when(jj > 0)
            def _wait_s1():
                pltpu.make_async_copy(buf1, acc.at[coli.at[j - 1]],
                                      sems1).wait()

            pltpu.async_copy(s_hbm.at[rowi.at[j + 1]], buf1, sem1)
            pltpu.make_async_copy(s_hbm.at[rowi.at[j]], buf0, sem0).wait()
            pltpu.async_copy(buf0, acc.at[coli.at[j]], sems0, add=True)

            @pl.when(j + 2 < k_ops)
            def _prefetch():
                pltpu.make_async_copy(buf0, acc.at[coli.at[j]], sems0).wait()
                pltpu.async_copy(s_hbm.at[rowi.at[j + 2]], buf0, sem0)

            pltpu.make_async_copy(s_hbm.at[rowi.at[j + 1]], buf1, sem1).wait()
            pltpu.async_copy(buf1, acc.at[coli.at[j + 1]], sems1, add=True)
            return _

        lax.fori_loop(0, khalf, edge, None)
        pltpu.make_async_copy(buf0, acc.at[coli.at[k_ops - 2]], sems0).wait()
        pltpu.make_async_copy(buf1, acc.at[coli.at[k_ops - 1]], sems1).wait()
        plsc.subcore_barrier()
        pltpu.sync_copy(acc.at[pl.ds(sid * rpt, rpt)],
                        out_hbm.at[cid, pl.ds(sid * rpt, rpt)])

    return kfn(s, row3, col3)


def _rows(c):
    return pl.BlockSpec((_R4, c), lambda i: (i, 0))


def _const(shape):
    return pl.BlockSpec(shape, lambda i: tuple(0 for _ in shape))


def _f32(shape):
    return jax.ShapeDtypeStruct(shape, jnp.float32)


_XE = np.zeros((128, 128), np.float32)
_XO = np.zeros((128, 128), np.float32)
for _i in range(4):
    for _c in range(32):
        _XE[16 * _i, 32 * _i + _c] = 1.0
        _XO[64 + 16 * _i, 32 * _i + _c] = 1.0


def _tc_dinv12(dp, xe, xo):
    m_rows = _N12 // 4
    qin = _R4 // 2

    def body(d_ref, xe_ref, xo_ref, o_ref):
        deg = d_ref[0] + d_ref[1]
        t = jnp.broadcast_to(deg[:, None, :], (qin, 2, 128)).reshape(_R4, 128)
        riota = lax.broadcasted_iota(jnp.int32, (_R4, 128), 0)
        val = jnp.where(
            riota % 2 == 0,
            jnp.dot(t, xe_ref[...], preferred_element_type=jnp.float32),
            jnp.dot(t, xo_ref[...], preferred_element_type=jnp.float32))
        o_ref[...] = lax.rsqrt(1.0 + val)

    return pl.pallas_call(
        body,
        grid=(m_rows // _R4,),
        in_specs=[pl.BlockSpec((2, qin, 128), lambda i: (0, i, 0)),
                  _const((128, 128)), _const((128, 128))],
        out_specs=_rows(128),
        out_shape=_f32((m_rows, 128)),
    )(dp, xe, xo)


def _tc_head(xp, dp0, fc1_Wb, fc1_bt, W1b, xe, xo):
    qin = _R4 // 2

    def body(x_ref, d_ref, fw_ref, fb_ref, w1_ref, xe_ref, xo_ref,
             s_ref, di_ref):
        deg = d_ref[0] + d_ref[1]
        t = jnp.broadcast_to(deg[:, None, :], (qin, 2, 128)).reshape(_R4, 128)
        riota = lax.broadcasted_iota(jnp.int32, (_R4, 128), 0)
        val = jnp.where(
            riota % 2 == 0,
            jnp.dot(t, xe_ref[...], preferred_element_type=jnp.float32),
            jnp.dot(t, xo_ref[...], preferred_element_type=jnp.float32))
        dinv = lax.rsqrt(1.0 + val)
        h = jnp.dot(x_ref[...], fw_ref[...], preferred_element_type=jnp.float32)
        h = jnp.maximum(h + fb_ref[...], 0.0)
        s_ref[...] = jnp.dot(h, w1_ref[...], preferred_element_type=jnp.float32) * dinv
        di_ref[...] = dinv

    return pl.pallas_call(
        body,
        grid=(_M0 // _R4,),
        in_specs=[_rows(16), pl.BlockSpec((2, qin, 128), lambda i: (0, i, 0)),
                  _const((16, 128)), _const((1, 128)), _const((128, 128)),
                  _const((128, 128)), _const((128, 128))],
        out_specs=[_rows(128), _rows(128)],
        out_shape=[_f32((_M0, 128)), _f32((_M0, 128))],
    )(xp, dp0, fc1_Wb, fc1_bt, W1b, xe, xo)


def _tc_scale(inp, dinv, Wb, m_rows):
    def body(x_ref, di_ref, w_ref, s_ref):
        s_ref[...] = jnp.dot(x_ref[...], w_ref[...], preferred_element_type=jnp.float32) * di_ref[...]

    return pl.pallas_call(
        body,
        grid=(m_rows // _R4,),
        in_specs=[_rows(128), _rows(128), _const((128, 128))],
        out_specs=_rows(128),
        out_shape=_f32((m_rows, 128)),
    )(inp, dinv, Wb)


_EJ = np.zeros((4, 128, 128), np.float32)
_OJ = np.zeros((4, 128, 128), np.float32)
for _j in range(4):
    for _b in (0, 1):
        for _u in (0, 1):
            for _cc in range(16):
                _EJ[_j, 32 * _j + 16 * _b + _cc, 64 * _u + 32 * _b + 2 * _cc] = 1.0
                _OJ[_j, 32 * _j + 16 * _b + _cc, 64 * _u + 32 * _b + 2 * _cc + 1] = 1.0


def _tc_up_res_scale(a, c, dinv, Wb, EW, OW, m_out, aligned, off=0):
    q = m_out // 8

    def body(a_ref, c_ref, di_ref, w_ref, ew_ref, ow_ref, s_ref):
        xa = c_ref[0:q, :]
        if aligned:
            xb = c_ref[off:off + q, :]
        else:
            xb = jnp.concatenate(
                [c_ref[q - 1:2 * q - 1, 64:128], c_ref[q:2 * q, 0:64]],
                axis=1)
        ys = [
            jnp.dot(xa, ew_ref[j], preferred_element_type=jnp.float32)
            + jnp.dot(xb, ow_ref[j], preferred_element_type=jnp.float32)
            for j in range(4)
        ]
        st = jnp.stack(ys, axis=1)
        st = jnp.broadcast_to(st[:, :, None, :], (q, 4, 2, 128))
        up = st.reshape(m_out, 128)
        base = jnp.dot(a_ref[...], w_ref[...], preferred_element_type=jnp.float32)
        s_ref[...] = (base + up) * di_ref[...]

    full = lambda shape: pl.BlockSpec(shape, lambda: tuple(0 for _ in shape))
    return pl.pallas_call(
        body,
        in_specs=[full((m_out, 128)), full((m_out // 4, 128)),
                  full((m_out, 128)), full((128, 128)),
                  full((4, 128, 128)), full((4, 128, 128))],
        out_specs=full((m_out, 128)),
        out_shape=_f32((m_out, 128)),
    )(a, c, dinv, Wb, EW, OW)


def _tc_combine(ap, s, dinv, bt, m_rows):
    def body(a_ref, s_ref, di_ref, b_ref, o_ref):
        acc = a_ref[0] + a_ref[1] + s_ref[...]
        o_ref[...] = jnp.maximum(di_ref[...] * acc + b_ref[...], 0.0)

    return pl.pallas_call(
        body,
        grid=(m_rows // _R4,),
        in_specs=[pl.BlockSpec((2, _R4, 128), lambda i: (0, i, 0)),
                  _rows(128), _rows(128), _const((1, 128))],
        out_specs=_rows(128),
        out_shape=_f32((m_rows, 128)),
    )(ap, s, dinv, bt)


def _tc_tail(ap, s, dinv, bt, fc2_Wb, fc2_bt):
    def body(a_ref, s_ref, di_ref, b_ref, fw_ref, fb_ref, o_ref):
        acc = a_ref[0] + a_ref[1] + s_ref[...]
        e = jnp.maximum(di_ref[...] * acc + b_ref[...], 0.0)
        o_ref[...] = lax.dot_general(
            fw_ref[...], e, (((0,), (1,)), ((), ())),
            preferred_element_type=jnp.float32) + fb_ref[...]

    rt = 2944

    return pl.pallas_call(
        body,
        grid=(_M0 // rt,),
        in_specs=[pl.BlockSpec((2, rt, 128), lambda i: (0, i, 0)),
                  pl.BlockSpec((rt, 128), lambda i: (i, 0)),
                  pl.BlockSpec((rt, 128), lambda i: (i, 0)),
                  _const((1, 128)), _const((128, 12)), _const((12, 1))],
        out_specs=pl.BlockSpec((12, rt), lambda i: (0, i)),
        out_shape=_f32((12, _M0)),
    )(ap, s, dinv, bt, fc2_Wb, fc2_bt)


def _pad_flat(v, total, fill):
    return jnp.concatenate(
        [v.astype(jnp.int32),
         jnp.full((total - v.shape[0],), fill, jnp.int32)])


def _down_packed(a_pack, nx, ny, m_pad):
    n = nx * ny
    v = a_pack[:n // 4].reshape(nx, ny // 4, 128)[::2]
    d = jnp.concatenate([v[..., 0:32], v[..., 64:96]], axis=-1)
    d = d.reshape(n // 16, 128)
    return jnp.pad(d, ((0, m_pad - n // 16), (0, 0)))


def _blockdiag(W):
    return jnp.kron(jnp.eye(4, dtype=jnp.float32), W)


def _tile4(b):
    return jnp.tile(b.reshape(1, -1), (1, 4))


def kernel(x, edge_index_0, edge_index_1, edge_index_2, index_0, index_1,
           index_2, fc1_W, fc1_b, conv1_W, conv1_b, conv2_W, conv2_b,
           conv3_W, conv3_b, conv4_W, conv4_b, conv5_W, conv5_b,
           fc2_W, fc2_b):
    del index_0, index_1, index_2
    rowf0 = _pad_flat(edge_index_0[0], _EP0, 0)
    colf0 = _pad_flat(edge_index_0[1], _EP0, _N0)
    rowf1 = _pad_flat(edge_index_1[0], _EP1, 0)
    colf1 = _pad_flat(edge_index_1[1], _EP1, _N1)
    rowf2 = _pad_flat(edge_index_2[0], _EP2, 0)
    colf2 = _pad_flat(edge_index_2[1], _EP2, _N2)
    row3_0 = rowf0.reshape(32, _K0, _CPE)
    col3_0 = colf0.reshape(32, _K0, _CPE)
    row3_1 = rowf1.reshape(32, _K1, _CPE)
    col3_1 = colf1.reshape(32, _K1, _CPE)
    row3_2 = rowf2.reshape(32, _K2, _CPE)
    col3_2 = colf2.reshape(32, _K2, _CPE)
    col12 = jnp.concatenate([colf1, colf2 + _NP1]).reshape(32, _KD, _CPE)

    W1b = _blockdiag(conv1_W)
    W2b = _blockdiag(conv2_W)
    W3b = _blockdiag(conv3_W)
    W4b = _blockdiag(conv4_W)
    W5b = _blockdiag(conv5_W)
    fc1_Wb = _blockdiag(fc1_W)
    fc2_Wb = _blockdiag(fc2_W)
    ej = jnp.asarray(_EJ)
    oj = jnp.asarray(_OJ)
    EW4 = jnp.einsum("jab,bc->jac", ej, W4b, precision=lax.Precision.HIGHEST)
    OW4 = jnp.einsum("jab,bc->jac", oj, W4b, precision=lax.Precision.HIGHEST)
    EW5 = jnp.einsum("jab,bc->jac", ej, W5b, precision=lax.Precision.HIGHEST)
    OW5 = jnp.einsum("jab,bc->jac", oj, W5b, precision=lax.Precision.HIGHEST)

    dp0, dp12 = _sc_deg(col3_0, col12)
    dinv12 = _tc_dinv12(dp12.reshape(2, _N12 // 8, 128),
                        jnp.asarray(_XE), jnp.asarray(_XO))
    dinv1 = dinv12[:_M1]
    dinv2 = dinv12[_M1:]

    xp = jnp.pad(x.reshape(_N0 // 4, 16), ((0, _M0 - _N0 // 4), (0, 0)))
    s1, dinv0 = _tc_head(xp, dp0.reshape(2, _NP0 // 8, 128), fc1_Wb,
                         _tile4(fc1_b), W1b, jnp.asarray(_XE),
                         jnp.asarray(_XO))
    ap = _sc_conv(s1.reshape(_NP0, 32), row3_0, col3_0, _NP0)
    A = _tc_combine(ap.reshape(2, _M0, 128), s1, dinv0, _tile4(conv1_b), _M0)

    B0 = _down_packed(A, _NX, _NY, _M1)
    s2 = _tc_scale(B0, dinv1, W2b, _M1)
    bp = _sc_conv(s2.reshape(_NP1, 32), row3_1, col3_1, _NP1)
    B = _tc_combine(bp.reshape(2, _M1, 128), s2, dinv1, _tile4(conv2_b), _M1)

    C0 = _down_packed(B, _NX // 2, _NY // 2, _M2)
    s3 = _tc_scale(C0, dinv2, W3b, _M2)
    cp = _sc_conv(s3.reshape(_NP2, 32), row3_2, col3_2, _NP2)
    C = _tc_combine(cp.reshape(2, _M2, 128), s3, dinv2, _tile4(conv3_b), _M2)

    s4 = _tc_up_res_scale(B, C, dinv1, W4b, EW4, OW4, _M1, aligned=False)
    dpp = _sc_conv(s4.reshape(_NP1, 32), row3_1, col3_1, _NP1)
    D = _tc_combine(dpp.reshape(2, _M1, 128), s4, dinv1, _tile4(conv4_b), _M1)

    s5 = _tc_up_res_scale(A, D, dinv0, W5b, EW5, OW5, _M0, aligned=True,
                          off=_N1 // 8)
    ep = _sc_conv(s5.reshape(_NP0, 32), row3_0, col3_0, _NP0)
    outT = _tc_tail(ep.reshape(2, _M0, 128), s5, dinv0, _tile4(conv5_b),
                    fc2_Wb, jnp.tile(fc2_b, 4).reshape(12, 1))
    out = outT.reshape(4, 3, _M0).transpose(1, 2, 0).reshape(3, _NP0)
    return out[:, :_N0].T
